# baseline (speedup 1.0000x reference)
import jax
import jax.numpy as jnp
from jax import lax
from jax.experimental import pallas as pl
from jax.experimental.pallas import tpu as pltpu
from jax.experimental.pallas import tpu_sc as plsc

B = 128
N = 32768
NC = 1
NS = 16
L = 16
NW = NC * NS
B_SC = 32
B_TC = B - B_SC
RPW = B_SC // NW
C = 4096
NCHUNK = N // C
UNROLL = 2
W = 256
K = N // W

_LN2 = 0.6931471805599453
_NLOG2E = -1.4426950408889634
_EPS = 1e-7
_LOG_COEF = (0.9998072558097292, -0.5017243070528501, 0.35092721871726945,
             -0.22560306077628844)
_SQRT2_2_BITS = 0x3F3504F3


def _vlog(y):
    bits = lax.bitcast_convert_type(y, jnp.int32)
    eb = bits - jnp.int32(_SQRT2_2_BITS)
    e = lax.shift_right_arithmetic(eb, jnp.int32(23))
    m = lax.bitcast_convert_type(
        lax.bitwise_and(eb, jnp.int32(0x007FFFFF)) + jnp.int32(_SQRT2_2_BITS),
        jnp.float32)
    ef = e.astype(jnp.float32)
    w = m - jnp.float32(1.0)
    p = jnp.float32(_LOG_COEF[3])
    for i in (2, 1, 0):
        p = p * w + jnp.float32(_LOG_COEF[i])
    return ef * jnp.float32(_LN2) + w * p


def _step(xv, hv, bv, cv, av):
    c = xv * hv
    s_inc = plsc.cumsum(c)
    z = cv + (s_inc - c) + bv
    t = xv * z
    u = jnp.exp(jnp.minimum(-t, jnp.float32(80.0)))
    yv = jnp.float32(1.0) / (jnp.float32(1.0) + u) + jnp.float32(_EPS)
    av = av + _vlog(yv)
    cv = cv + jnp.sum(c)
    return cv, av


def _body(x_hbm, ham_hbm, bias_hbm, beta_hbm, out_hbm, x_v, ham_v, bias_v,
          beta_v, out_v, sem0, sem1):
    wid = lax.axis_index("s") * NC + lax.axis_index("c")
    r0 = B_TC + wid * RPW
    sems = (sem0, sem1)

    pltpu.sync_copy(beta_hbm, beta_v)
    bb = beta_v[...]

    def start(k, p):
        col = k * C
        off = p * RPW * C
        offc = p * C
        d = [pltpu.async_copy(ham_hbm.at[pl.ds(col, C)],
                              ham_v.at[pl.ds(offc, C)], sems[p]),
             pltpu.async_copy(bias_hbm.at[pl.ds(col, C)],
                              bias_v.at[pl.ds(offc, C)], sems[p])]
        for r in range(RPW):
            d.append(pltpu.async_copy(x_hbm.at[r0 + r, pl.ds(col, C)],
                                      x_v.at[pl.ds(off + r * C, C)], sems[p]))
        return d

    carries = tuple(jnp.zeros((L,), jnp.float32) for _ in range(RPW))
    accs = tuple(jnp.zeros((L,), jnp.float32) for _ in range(RPW))

    pend = start(0, 0)
    for k in range(NCHUNK):
        p = k & 1
        for d in pend:
            d.wait()
        if k + 1 < NCHUNK:
            pend = start(k + 1, 1 - p)
        off = p * RPW * C
        offc = p * C

        def body(i, st):
            cvs, avs = st
            ncvs, navs = list(cvs), list(avs)
            for j in range(UNROLL):
                o = (i * UNROLL + j) * L
                hv = ham_v[pl.ds(offc + o, L)] * bb
                bv = bias_v[pl.ds(offc + o, L)]
                for r in range(RPW):
                    xv = x_v[pl.ds(off + o + r * C, L)]
                    ncvs[r], navs[r] = _step(xv, hv, bv, ncvs[r], navs[r])
            return tuple(ncvs), tuple(navs)

        carries, accs = lax.fori_loop(0, C // (L * UNROLL), body,
                                      (carries, accs))

    lanes = lax.iota(jnp.int32, L)
    out_vec = jnp.zeros((L,), jnp.float32)
    for r in range(RPW):
        out_vec = jnp.where(lanes == jnp.int32(r), jnp.sum(accs[r]), out_vec)
    out_v[...] = out_vec
    pltpu.sync_copy(out_v, out_hbm.at[wid])


@jax.jit
def _sc_call(x, ham, bias, beta_vec):
    fn = pl.kernel(
        _body,
        out_type=jax.ShapeDtypeStruct((NW, L), jnp.float32),
        mesh=plsc.VectorSubcoreMesh(core_axis_name="c", subcore_axis_name="s",
                                    num_cores=NC, num_subcores=NS),
        scratch_types=[
            pltpu.VMEM((2 * RPW * C,), jnp.float32),
            pltpu.VMEM((2 * C,), jnp.float32),
            pltpu.VMEM((2 * C,), jnp.float32),
            pltpu.VMEM((L,), jnp.float32),
            pltpu.VMEM((L,), jnp.float32),
            pltpu.SemaphoreType.DMA,
            pltpu.SemaphoreType.DMA,
        ],
        compiler_params=pltpu.CompilerParams(needs_layout_passes=False,
                                             disable_bounds_checks=True),
    )
    return fn(x, ham, bias, beta_vec)


def _tc_body(x_ref, hamb_ref, bias_ref, out_ref, carry_ref, acc_ref, u_ref):
    i = pl.program_id(0)

    @pl.when(i == 0)
    def _init():
        carry_ref[...] = jnp.zeros_like(carry_ref)
        acc_ref[...] = jnp.zeros_like(acc_ref)
        rows = lax.broadcasted_iota(jnp.int32, (W, W), 0)
        cols = lax.broadcasted_iota(jnp.int32, (W, W), 1)
        u_ref[...] = (rows < cols).astype(jnp.float32)

    xb = x_ref[...]
    c = xb * hamb_ref[...]
    h_loc = jax.lax.dot_general(c, u_ref[...], (((1,), (0,)), ((), ())),
                                preferred_element_type=jnp.float32)
    z = (carry_ref[:, 0:1] + h_loc) + bias_ref[...]
    t = xb * z
    lp = jnp.log(jax.nn.sigmoid(t) + jnp.float32(_EPS))
    acc_ref[...] += lp.reshape(B_TC, W // 128, 128).sum(axis=1)
    carry_ref[:, 0:1] += h_loc[:, W - 1:W] + c[:, W - 1:W]

    @pl.when(i == K - 1)
    def _fin():
        out_ref[...] = jnp.sum(acc_ref[...], axis=1, keepdims=True)


@jax.jit
def _tc_call(x, hamb2d, bias2d):
    return pl.pallas_call(
        _tc_body,
        grid=(K,),
        in_specs=[
            pl.BlockSpec((B_TC, W), lambda i: (0, i)),
            pl.BlockSpec((1, W), lambda i: (0, i)),
            pl.BlockSpec((1, W), lambda i: (0, i)),
        ],
        out_specs=pl.BlockSpec((B_TC, 1), lambda i: (0, 0)),
        out_shape=jax.ShapeDtypeStruct((B_TC, 1), jnp.float32),
        scratch_shapes=[
            pltpu.VMEM((B_TC, 128), jnp.float32),
            pltpu.VMEM((B_TC, 128), jnp.float32),
            pltpu.VMEM((W, W), jnp.float32),
        ],
        compiler_params=pltpu.CompilerParams(
            dimension_semantics=("arbitrary",)),
    )(x, hamb2d, bias2d)


def kernel(x, ham_params, bias, beta):
    beta_vec = jnp.full((L,), beta, jnp.float32)
    sc2d = _sc_call(x, ham_params, bias, beta_vec)
    hamb2d = (ham_params * beta).reshape(1, N)
    tc2d = _tc_call(x, hamb2d, bias.reshape(1, N))
    return jnp.concatenate([tc2d[:, 0], sc2d[:, :RPW].reshape(B_SC)])


# speedup vs baseline: 1.5463x; 1.5463x over previous
import jax
import jax.numpy as jnp
from jax import lax
from jax.experimental import pallas as pl
from jax.experimental.pallas import tpu as pltpu
from jax.experimental.pallas import tpu_sc as plsc

B = 128
N = 32768
NC = 1
NS = 16
L = 16
NW = NC * NS
B_SC = 48
B_TC = B - B_SC
RPW = B_SC // NW
C = 4096
NCHUNK = N // C
UNROLL = 2
W = 512
K = N // W

_LN2 = 0.6931471805599453
_NLOG2E = -1.4426950408889634
_EPS = 1e-7
_LOG_COEF = (0.9998072558097292, -0.5017243070528501, 0.35092721871726945,
             -0.22560306077628844)
_SQRT2_2_BITS = 0x3F3504F3


def _vlog(y):
    bits = lax.bitcast_convert_type(y, jnp.int32)
    eb = bits - jnp.int32(_SQRT2_2_BITS)
    e = lax.shift_right_arithmetic(eb, jnp.int32(23))
    m = lax.bitcast_convert_type(
        lax.bitwise_and(eb, jnp.int32(0x007FFFFF)) + jnp.int32(_SQRT2_2_BITS),
        jnp.float32)
    ef = e.astype(jnp.float32)
    w = m - jnp.float32(1.0)
    p = jnp.float32(_LOG_COEF[3])
    for i in (2, 1, 0):
        p = p * w + jnp.float32(_LOG_COEF[i])
    return ef * jnp.float32(_LN2) + w * p


def _step(xv, hv, bv, cv, av):
    c = xv * hv
    s_inc = plsc.cumsum(c)
    z = cv + (s_inc - c) + bv
    t = xv * z
    u = jnp.exp(jnp.minimum(-t, jnp.float32(80.0)))
    yv = jnp.float32(1.0) / (jnp.float32(1.0) + u) + jnp.float32(_EPS)
    av = av + _vlog(yv)
    cv = cv + jnp.sum(c)
    return cv, av


def _body(x_hbm, ham_hbm, bias_hbm, beta_hbm, out_hbm, x_v, ham_v, bias_v,
          beta_v, out_v, sem0, sem1):
    wid = lax.axis_index("s") * NC + lax.axis_index("c")
    r0 = B_TC + wid * RPW
    sems = (sem0, sem1)

    pltpu.sync_copy(beta_hbm, beta_v)
    bb = beta_v[...]

    def start(k, p):
        col = k * C
        off = p * RPW * C
        offc = p * C
        d = [pltpu.async_copy(ham_hbm.at[pl.ds(col, C)],
                              ham_v.at[pl.ds(offc, C)], sems[p]),
             pltpu.async_copy(bias_hbm.at[pl.ds(col, C)],
                              bias_v.at[pl.ds(offc, C)], sems[p])]
        for r in range(RPW):
            d.append(pltpu.async_copy(x_hbm.at[r0 + r, pl.ds(col, C)],
                                      x_v.at[pl.ds(off + r * C, C)], sems[p]))
        return d

    carries = tuple(jnp.zeros((L,), jnp.float32) for _ in range(RPW))
    accs = tuple(jnp.zeros((L,), jnp.float32) for _ in range(RPW))

    pend = start(0, 0)
    for k in range(NCHUNK):
        p = k & 1
        for d in pend:
            d.wait()
        if k + 1 < NCHUNK:
            pend = start(k + 1, 1 - p)
        off = p * RPW * C
        offc = p * C

        def body(i, st):
            cvs, avs = st
            ncvs, navs = list(cvs), list(avs)
            for j in range(UNROLL):
                o = (i * UNROLL + j) * L
                hv = ham_v[pl.ds(offc + o, L)] * bb
                bv = bias_v[pl.ds(offc + o, L)]
                for r in range(RPW):
                    xv = x_v[pl.ds(off + o + r * C, L)]
                    ncvs[r], navs[r] = _step(xv, hv, bv, ncvs[r], navs[r])
            return tuple(ncvs), tuple(navs)

        carries, accs = lax.fori_loop(0, C // (L * UNROLL), body,
                                      (carries, accs))

    lanes = lax.iota(jnp.int32, L)
    out_vec = jnp.zeros((L,), jnp.float32)
    for r in range(RPW):
        out_vec = jnp.where(lanes == jnp.int32(r), jnp.sum(accs[r]), out_vec)
    out_v[...] = out_vec
    pltpu.sync_copy(out_v, out_hbm.at[wid])


@jax.jit
def _sc_call(x, ham, bias, beta_vec):
    fn = pl.kernel(
        _body,
        out_type=jax.ShapeDtypeStruct((NW, L), jnp.float32),
        mesh=plsc.VectorSubcoreMesh(core_axis_name="c", subcore_axis_name="s",
                                    num_cores=NC, num_subcores=NS),
        scratch_types=[
            pltpu.VMEM((2 * RPW * C,), jnp.float32),
            pltpu.VMEM((2 * C,), jnp.float32),
            pltpu.VMEM((2 * C,), jnp.float32),
            pltpu.VMEM((L,), jnp.float32),
            pltpu.VMEM((L,), jnp.float32),
            pltpu.SemaphoreType.DMA,
            pltpu.SemaphoreType.DMA,
        ],
        compiler_params=pltpu.CompilerParams(needs_layout_passes=False,
                                             disable_bounds_checks=True),
    )
    return fn(x, ham, bias, beta_vec)


def _tc_body(x_ref, hamb_ref, bias_ref, out_ref, carry_ref, acc_ref, u_ref):
    i = pl.program_id(0)

    @pl.when(i == 0)
    def _init():
        carry_ref[...] = jnp.zeros_like(carry_ref)
        acc_ref[...] = jnp.zeros_like(acc_ref)
        rows = lax.broadcasted_iota(jnp.int32, (W, W), 0)
        cols = lax.broadcasted_iota(jnp.int32, (W, W), 1)
        u_ref[...] = (rows < cols).astype(jnp.float32)

    xb = x_ref[...]
    c = xb * hamb_ref[...]
    h_loc = jax.lax.dot_general(c, u_ref[...], (((1,), (0,)), ((), ())),
                                preferred_element_type=jnp.float32)
    z = (carry_ref[:, 0:1] + h_loc) + bias_ref[...]
    t = xb * z
    lp = jnp.log(jax.nn.sigmoid(t) + jnp.float32(_EPS))
    acc_ref[...] += lp.reshape(B_TC, W // 128, 128).sum(axis=1)
    carry_ref[:, 0:1] += h_loc[:, W - 1:W] + c[:, W - 1:W]

    @pl.when(i == K - 1)
    def _fin():
        out_ref[...] = jnp.sum(acc_ref[...], axis=1, keepdims=True)


@jax.jit
def _tc_call(x, hamb2d, bias2d):
    return pl.pallas_call(
        _tc_body,
        grid=(K,),
        in_specs=[
            pl.BlockSpec((B_TC, W), lambda i: (0, i)),
            pl.BlockSpec((1, W), lambda i: (0, i)),
            pl.BlockSpec((1, W), lambda i: (0, i)),
        ],
        out_specs=pl.BlockSpec((B_TC, 1), lambda i: (0, 0)),
        out_shape=jax.ShapeDtypeStruct((B_TC, 1), jnp.float32),
        scratch_shapes=[
            pltpu.VMEM((B_TC, 128), jnp.float32),
            pltpu.VMEM((B_TC, 128), jnp.float32),
            pltpu.VMEM((W, W), jnp.float32),
        ],
        compiler_params=pltpu.CompilerParams(
            dimension_semantics=("arbitrary",)),
    )(x, hamb2d, bias2d)


def kernel(x, ham_params, bias, beta):
    beta_vec = jnp.full((L,), beta, jnp.float32)
    sc2d = _sc_call(x, ham_params, bias, beta_vec)
    hamb2d = (ham_params * beta).reshape(1, N)
    tc2d = _tc_call(x, hamb2d, bias.reshape(1, N))
    return jnp.concatenate([tc2d[:, 0], sc2d[:, :RPW].reshape(B_SC)])


# speedup vs baseline: 1.5518x; 1.0036x over previous
import jax
import jax.numpy as jnp
from jax import lax
from jax.experimental import pallas as pl
from jax.experimental.pallas import tpu as pltpu
from jax.experimental.pallas import tpu_sc as plsc

B = 128
N = 32768
NC = 1
NS = 16
L = 16
NW = NC * NS
B_SC = 48
B_TC = B - B_SC
RPW = B_SC // NW
C = 4096
NCHUNK = N // C
UNROLL = 2
W = 512
K = N // W

_LN2 = 0.6931471805599453
_NLOG2E = -1.4426950408889634
_EPS = 1e-7
_LOG_COEF = (0.9998072558097292, -0.5017243070528501, 0.35092721871726945,
             -0.22560306077628844)
_SQRT2_2_BITS = 0x3F3504F3


def _vlog(y):
    bits = lax.bitcast_convert_type(y, jnp.int32)
    eb = bits - jnp.int32(_SQRT2_2_BITS)
    e = lax.shift_right_arithmetic(eb, jnp.int32(23))
    m = lax.bitcast_convert_type(
        lax.bitwise_and(eb, jnp.int32(0x007FFFFF)) + jnp.int32(_SQRT2_2_BITS),
        jnp.float32)
    ef = e.astype(jnp.float32)
    w = m - jnp.float32(1.0)
    p = jnp.float32(_LOG_COEF[3])
    for i in (2, 1, 0):
        p = p * w + jnp.float32(_LOG_COEF[i])
    return ef * jnp.float32(_LN2) + w * p


def _step(xv, hv, bv, cv, av):
    c = xv * hv
    s_inc = plsc.cumsum(c)
    z = cv + (s_inc - c) + bv
    t = xv * z
    u = jnp.exp(jnp.minimum(-t, jnp.float32(80.0)))
    yv = jnp.float32(1.0) / (jnp.float32(1.0) + u) + jnp.float32(_EPS)
    av = av + _vlog(yv)
    cv = cv + jnp.sum(c)
    return cv, av


def _body(x_hbm, ham_hbm, bias_hbm, beta_hbm, out_hbm, x_v, ham_v, bias_v,
          beta_v, out_v, sem0, sem1):
    wid = lax.axis_index("s") * NC + lax.axis_index("c")
    r0 = B_TC + wid * RPW
    sems = (sem0, sem1)

    pltpu.sync_copy(beta_hbm, beta_v)
    bb = beta_v[...]

    def start(k, p):
        col = k * C
        off = p * RPW * C
        offc = p * C
        d = [pltpu.async_copy(ham_hbm.at[pl.ds(col, C)],
                              ham_v.at[pl.ds(offc, C)], sems[p]),
             pltpu.async_copy(bias_hbm.at[pl.ds(col, C)],
                              bias_v.at[pl.ds(offc, C)], sems[p])]
        for r in range(RPW):
            d.append(pltpu.async_copy(x_hbm.at[r0 + r, pl.ds(col, C)],
                                      x_v.at[pl.ds(off + r * C, C)], sems[p]))
        return d

    carries = tuple(jnp.zeros((L,), jnp.float32) for _ in range(RPW))
    accs = tuple(jnp.zeros((L,), jnp.float32) for _ in range(RPW))

    pend = start(0, 0)
    for k in range(NCHUNK):
        p = k & 1
        for d in pend:
            d.wait()
        if k + 1 < NCHUNK:
            pend = start(k + 1, 1 - p)
        off = p * RPW * C
        offc = p * C

        def body(i, st):
            cvs, avs = st
            ncvs, navs = list(cvs), list(avs)
            for j in range(UNROLL):
                o = (i * UNROLL + j) * L
                hv = ham_v[pl.ds(offc + o, L)] * bb
                bv = bias_v[pl.ds(offc + o, L)]
                for r in range(RPW):
                    xv = x_v[pl.ds(off + o + r * C, L)]
                    ncvs[r], navs[r] = _step(xv, hv, bv, ncvs[r], navs[r])
            return tuple(ncvs), tuple(navs)

        carries, accs = lax.fori_loop(0, C // (L * UNROLL), body,
                                      (carries, accs))

    lanes = lax.iota(jnp.int32, L)
    out_vec = jnp.zeros((L,), jnp.float32)
    for r in range(RPW):
        out_vec = jnp.where(lanes == jnp.int32(r), jnp.sum(accs[r]), out_vec)
    out_v[...] = out_vec
    pltpu.sync_copy(out_v, out_hbm.at[wid])


@jax.jit
def _sc_call(x, ham, bias, beta_vec):
    fn = pl.kernel(
        _body,
        out_type=jax.ShapeDtypeStruct((NW, L), jnp.float32),
        mesh=plsc.VectorSubcoreMesh(core_axis_name="c", subcore_axis_name="s",
                                    num_cores=NC, num_subcores=NS),
        scratch_types=[
            pltpu.VMEM((2 * RPW * C,), jnp.float32),
            pltpu.VMEM((2 * C,), jnp.float32),
            pltpu.VMEM((2 * C,), jnp.float32),
            pltpu.VMEM((L,), jnp.float32),
            pltpu.VMEM((L,), jnp.float32),
            pltpu.SemaphoreType.DMA,
            pltpu.SemaphoreType.DMA,
        ],
        compiler_params=pltpu.CompilerParams(needs_layout_passes=False,
                                             disable_bounds_checks=True),
    )
    return fn(x, ham, bias, beta_vec)


def _tc_body(x_ref, hamb_ref, bias_ref, out_ref, carry_ref, acc_ref, u_ref):
    i = pl.program_id(0)

    @pl.when(i == 0)
    def _init():
        carry_ref[...] = jnp.zeros_like(carry_ref)
        acc_ref[...] = jnp.zeros_like(acc_ref)
        rows = lax.broadcasted_iota(jnp.int32, (W, W), 0)
        cols = lax.broadcasted_iota(jnp.int32, (W, W), 1)
        u_ref[...] = (rows < cols).astype(jnp.bfloat16)

    xb = x_ref[...]
    c = xb * hamb_ref[...]
    h_loc = jax.lax.dot_general(c.astype(jnp.bfloat16), u_ref[...],
                                (((1,), (0,)), ((), ())),
                                preferred_element_type=jnp.float32)
    z = (carry_ref[:, 0:1] + h_loc) + bias_ref[...]
    t = xb * z
    lp = jnp.log(jax.nn.sigmoid(t) + jnp.float32(_EPS))
    acc_ref[...] += lp.reshape(B_TC, W // 128, 128).sum(axis=1)
    carry_ref[:, 0:1] += jnp.sum(c, axis=1, keepdims=True)

    @pl.when(i == K - 1)
    def _fin():
        out_ref[...] = jnp.sum(acc_ref[...], axis=1, keepdims=True)


@jax.jit
def _tc_call(x, hamb2d, bias2d):
    return pl.pallas_call(
        _tc_body,
        grid=(K,),
        in_specs=[
            pl.BlockSpec((B_TC, W), lambda i: (0, i)),
            pl.BlockSpec((1, W), lambda i: (0, i)),
            pl.BlockSpec((1, W), lambda i: (0, i)),
        ],
        out_specs=pl.BlockSpec((B_TC, 1), lambda i: (0, 0)),
        out_shape=jax.ShapeDtypeStruct((B_TC, 1), jnp.float32),
        scratch_shapes=[
            pltpu.VMEM((B_TC, 128), jnp.float32),
            pltpu.VMEM((B_TC, 128), jnp.float32),
            pltpu.VMEM((W, W), jnp.bfloat16),
        ],
        compiler_params=pltpu.CompilerParams(
            dimension_semantics=("arbitrary",)),
    )(x, hamb2d, bias2d)


def kernel(x, ham_params, bias, beta):
    beta_vec = jnp.full((L,), beta, jnp.float32)
    sc2d = _sc_call(x, ham_params, bias, beta_vec)
    hamb2d = (ham_params * beta).reshape(1, N)
    tc2d = _tc_call(x, hamb2d, bias.reshape(1, N))
    return jnp.concatenate([tc2d[:, 0], sc2d[:, :RPW].reshape(B_SC)])
